# baseline (speedup 1.0000x reference)
import functools

import jax
import jax.numpy as jnp
from jax import lax
from jax.experimental import pallas as pl
from jax.experimental.pallas import tpu as pltpu
from jax.experimental.pallas import tpu_sc as plsc

NC = 2
NS = 16
NW = NC * NS
NSEG = 6
D = 128


def _mesh():
    return plsc.VectorSubcoreMesh(
        core_axis_name="c", subcore_axis_name="s",
        num_cores=NC, num_subcores=NS)


def _segments(total, nseg):
    bounds = [((total * i // nseg) // 8) * 8 for i in range(nseg + 1)]
    bounds[-1] = total
    return [(bounds[i], bounds[i + 1] - bounds[i]) for i in range(nseg)]


def _make_deg(nrows, ept, nseg=NSEG):
    rpt = nrows // NS
    segs = _segments(ept, nseg)
    seg_max = max(sz for _, sz in segs)

    def body(dsts, zeros, ones, out, idxd, buf, accum, ssem):
        c = lax.axis_index("c")
        s = lax.axis_index("s")
        wid = s * NC + c
        pltpu.sync_copy(dsts.at[wid], idxd)
        pltpu.sync_copy(ones, buf)
        pltpu.sync_copy(zeros.at[pl.ds(s * rpt, rpt)],
                        accum.at[pl.ds(s * rpt, rpt)])
        plsc.subcore_barrier()
        descs = [
            pltpu.async_copy(buf.at[pl.ds(0, sz)],
                             accum.at[idxd.at[pl.ds(a, sz)]],
                             ssem, add=True)
            for a, sz in segs
        ]
        for d in descs:
            d.wait()
        plsc.subcore_barrier()
        pltpu.sync_copy(accum.at[pl.ds(s * rpt, rpt)],
                        out.at[c, pl.ds(s * rpt, rpt)])

    return pl.kernel(
        body,
        out_type=jax.ShapeDtypeStruct((NC, nrows, 16), jnp.float32),
        mesh=_mesh(),
        compiler_params=pltpu.CompilerParams(use_tc_tiling_on_sc=False),
        scratch_types=[
            pltpu.VMEM((ept,), jnp.int32),
            pltpu.VMEM((seg_max, 16), jnp.float32),
            pltpu.VMEM_SHARED((nrows, 16), jnp.float32),
            pltpu.SemaphoreType.DMA,
        ],
    )


def _make_agg(nrows, ept, f, nseg=NSEG):
    rpt = nrows // NS
    segs = _segments(ept, nseg)
    seg_max = max(sz for _, sz in segs)

    def body(srcs, dsts, g, zeros, out, idxs, idxd, buf0, buf1, accum,
             gsem, ssem):
        c = lax.axis_index("c")
        s = lax.axis_index("s")
        wid = s * NC + c
        bufs = [buf0, buf1]
        pltpu.sync_copy(srcs.at[wid], idxs)
        pltpu.sync_copy(dsts.at[wid], idxd)

        def gather(i):
            a, sz = segs[i]
            return pltpu.async_copy(
                g.at[idxs.at[pl.ds(a, sz)]],
                bufs[i % 2].at[pl.ds(0, sz)], gsem)

        def scatter(i):
            a, sz = segs[i]
            return pltpu.async_copy(
                bufs[i % 2].at[pl.ds(0, sz)],
                accum.at[idxd.at[pl.ds(a, sz)]],
                ssem, add=True)

        gd = gather(0)
        pltpu.sync_copy(zeros.at[pl.ds(s * rpt, rpt)],
                        accum.at[pl.ds(s * rpt, rpt)])
        plsc.subcore_barrier()

        sds = [None] * nseg
        for i in range(nseg):
            gd.wait()
            sds[i] = scatter(i)
            if i + 1 < nseg:
                if i >= 1:
                    sds[i - 1].wait()
                gd = gather(i + 1)
        for i in range(max(0, nseg - 2), nseg):
            sds[i].wait()

        plsc.subcore_barrier()
        pltpu.sync_copy(accum.at[pl.ds(s * rpt, rpt)],
                        out.at[c, pl.ds(s * rpt, rpt)])

    return pl.kernel(
        body,
        out_type=jax.ShapeDtypeStruct((NC, nrows, f), jnp.float32),
        mesh=_mesh(),
        compiler_params=pltpu.CompilerParams(use_tc_tiling_on_sc=False),
        scratch_types=[
            pltpu.VMEM((ept,), jnp.int32),
            pltpu.VMEM((ept,), jnp.int32),
            pltpu.VMEM((seg_max, f), jnp.float32),
            pltpu.VMEM((seg_max, f), jnp.float32),
            pltpu.VMEM_SHARED((nrows, f), jnp.float32),
            pltpu.SemaphoreType.DMA,
            pltpu.SemaphoreType.DMA,
        ],
    )


def _tc_a0(x_ref, w1_ref, xw_ref):
    xw_ref[...] = jnp.dot(x_ref[...], w1_ref[...],
                          preferred_element_type=jnp.float32)


def _tc_a1(n, br, deg_ref, xw_ref, g1_ref, dinv_ref):
    i = pl.program_id(0)
    dp = deg_ref[...]
    deg = dp[0] + dp[1] + 1.0
    rows = i * br + lax.broadcasted_iota(jnp.int32, (br, 16), 0)
    dinv = jnp.where(rows < n, lax.rsqrt(deg), 0.0)
    g1_ref[...] = xw_ref[...] * dinv
    dinv_ref[...] = dinv


def _tc_b(s1_ref, g1_ref, dinv_ref, b1_ref, w2_ref, g2_ref):
    sp = s1_ref[...]
    dinv = dinv_ref[...]
    h1 = jnp.maximum((sp[0] + sp[1] + g1_ref[...]) * dinv + b1_ref[...], 0.0)
    g2_ref[...] = jnp.dot(h1, w2_ref[...],
                          preferred_element_type=jnp.float32) * dinv


def _tc_c(s2_ref, g2_ref, dinv_ref, b2_ref, w3_ref, h2_ref, g3_ref):
    sp = s2_ref[...]
    dinv = dinv_ref[...]
    hraw = (sp[0] + sp[1] + g2_ref[...]) * dinv + b2_ref[...]
    nrm = jnp.sqrt(jnp.sum(hraw * hraw, axis=1, keepdims=True))
    h2 = hraw / jnp.maximum(nrm, 1e-12)
    h2_ref[...] = h2
    g3_ref[...] = jnp.dot(h2, w3_ref[...],
                          preferred_element_type=jnp.float32) * dinv


def _tc_d(n, br, nb, s3_ref, g3_ref, dinv_ref, b3_ref, wp_ref, bp_ref,
          wl_ref, bl_ref, ls_ref, hp_ref, acc_ref):
    i = pl.program_id(0)
    sp = s3_ref[...]
    dinv = dinv_ref[...]
    h3 = (sp[0] + sp[1] + g3_ref[...]) * dinv + b3_ref[...]
    hp = jnp.dot(h3, wp_ref[...], preferred_element_type=jnp.float32) \
        + bp_ref[...]
    hp_ref[...] = hp
    rows = i * br + lax.broadcasted_iota(jnp.int32, (br, 16), 0)
    part = jnp.sum(jnp.where(rows < n, hp, 0.0), axis=0, keepdims=True)

    @pl.when(i == 0)
    def _():
        acc_ref[...] = part

    @pl.when(i > 0)
    def _():
        acc_ref[...] += part

    @pl.when(i == nb - 1)
    def _():
        pooled = acc_ref[...] * (1.0 / n)
        logits = jnp.dot(pooled, wl_ref[...],
                         preferred_element_type=jnp.float32) + bl_ref[...]
        m = jnp.max(logits, axis=1, keepdims=True)
        e2 = logits - m
        ls_ref[...] = e2 - jnp.log(jnp.sum(jnp.exp(e2), axis=1, keepdims=True))


def kernel(x, edge_index, W1, b1, W2, b2, W3, b3, Wp, bp, Wl, bl):
    f32 = jnp.float32
    n = x.shape[0]
    e = edge_index.shape[1]
    nrows = ((n + 1 + 127) // 128) * 128
    br = nrows // 16
    nb = nrows // br
    align = NW * 8
    epad = ((e + align - 1) // align) * align
    ept = epad // NW
    seg_max = max(sz for _, sz in _segments(ept, NSEG))

    ei = edge_index.astype(jnp.int32)
    if epad != e:
        ei = jnp.concatenate(
            [ei, jnp.full((2, epad - e), n, jnp.int32)], axis=1)
    src_p = ei[0].reshape(NW, ept)
    dst_p = ei[1].reshape(NW, ept)

    x_p = jnp.zeros((nrows, D), f32).at[:n].set(x)
    w2p = jnp.zeros((16, 16), f32).at[:, :3].set(W2)
    w3p = jnp.zeros((16, 16), f32).at[:3, :3].set(W3)
    wpp = jnp.zeros((16, 16), f32).at[:3, :3].set(Wp)
    wlp = jnp.zeros((16, Wl.shape[1]), f32).at[:3].set(Wl)
    b1r = b1.reshape(1, 16)
    b2p = jnp.zeros((1, 16), f32).at[0, :3].set(b2)
    b3p = jnp.zeros((1, 16), f32).at[0, :3].set(b3)
    bpp = jnp.zeros((1, 16), f32).at[0, :3].set(bp)
    blr = bl.reshape(1, -1)

    z16 = jnp.zeros((nrows, 16), f32)
    ones = jnp.ones((seg_max, 16), f32)

    row2 = lambda i: (0, i, 0)
    row = lambda i: (i, 0)
    fixed = lambda i: (0, 0)
    sp2 = pl.BlockSpec((2, br, 16), row2)
    spg = pl.BlockSpec((br, 16), row)

    xw = pl.pallas_call(
        _tc_a0,
        grid=(nb,),
        in_specs=[pl.BlockSpec((br, D), row), pl.BlockSpec((D, 16), fixed)],
        out_specs=spg,
        out_shape=jax.ShapeDtypeStruct((nrows, 16), f32),
    )(x_p, W1)

    degp = _make_deg(nrows, ept)(dst_p, z16, ones)

    g1, dinv = pl.pallas_call(
        functools.partial(_tc_a1, n, br),
        grid=(nb,),
        in_specs=[sp2, spg],
        out_specs=[spg, spg],
        out_shape=[jax.ShapeDtypeStruct((nrows, 16), f32),
                   jax.ShapeDtypeStruct((nrows, 16), f32)],
    )(degp, xw)

    s1 = _make_agg(nrows, ept, 16)(src_p, dst_p, g1, z16)

    g2 = pl.pallas_call(
        _tc_b,
        grid=(nb,),
        in_specs=[sp2, spg, spg, pl.BlockSpec((1, 16), fixed),
                  pl.BlockSpec((16, 16), fixed)],
        out_specs=spg,
        out_shape=jax.ShapeDtypeStruct((nrows, 16), f32),
    )(s1, g1, dinv, b1r, w2p)

    s2 = _make_agg(nrows, ept, 16)(src_p, dst_p, g2, z16)

    h2, g3 = pl.pallas_call(
        _tc_c,
        grid=(nb,),
        in_specs=[sp2, spg, spg, pl.BlockSpec((1, 16), fixed),
                  pl.BlockSpec((16, 16), fixed)],
        out_specs=[spg, spg],
        out_shape=[jax.ShapeDtypeStruct((nrows, 16), f32),
                   jax.ShapeDtypeStruct((nrows, 16), f32)],
    )(s2, g2, dinv, b2p, w3p)

    s3 = _make_agg(nrows, ept, 16)(src_p, dst_p, g3, z16)

    ls, hp = pl.pallas_call(
        functools.partial(_tc_d, n, br, nb),
        grid=(nb,),
        in_specs=[sp2, spg, spg, pl.BlockSpec((1, 16), fixed),
                  pl.BlockSpec((16, 16), fixed),
                  pl.BlockSpec((1, 16), fixed),
                  pl.BlockSpec((16, Wl.shape[1]), fixed),
                  pl.BlockSpec((1, Wl.shape[1]), fixed)],
        out_specs=[pl.BlockSpec((1, Wl.shape[1]), fixed), spg],
        out_shape=[jax.ShapeDtypeStruct((1, Wl.shape[1]), f32),
                   jax.ShapeDtypeStruct((nrows, 16), f32)],
        scratch_shapes=[pltpu.VMEM((1, 16), f32)],
    )(s3, g3, dinv, b3p, wpp, bpp, wlp, blr)

    return (ls, (hp[:n, :3], h2[:n, :3]))


# speedup vs baseline: 1.1327x; 1.1327x over previous
import functools

import jax
import jax.numpy as jnp
from jax import lax
from jax.experimental import pallas as pl
from jax.experimental.pallas import tpu as pltpu
from jax.experimental.pallas import tpu_sc as plsc

NC = 2
NS = 16
NW = NC * NS
NSEG = 4
D = 128


def _mesh():
    return plsc.VectorSubcoreMesh(
        core_axis_name="c", subcore_axis_name="s",
        num_cores=NC, num_subcores=NS)


def _segments(total, nseg):
    bounds = [((total * i // nseg) // 8) * 8 for i in range(nseg + 1)]
    bounds[-1] = total
    return [(bounds[i], bounds[i + 1] - bounds[i]) for i in range(nseg)]


def _make_deg(nrows, ept, nseg=NSEG):
    rpt = nrows // NS
    segs = _segments(ept, nseg)
    seg_max = max(sz for _, sz in segs)

    def body(dsts, zeros, ones, out, idxd, buf, accum, ssem):
        c = lax.axis_index("c")
        s = lax.axis_index("s")
        wid = s * NC + c
        pltpu.sync_copy(dsts.at[wid], idxd)
        pltpu.sync_copy(ones, buf)
        pltpu.sync_copy(zeros.at[pl.ds(s * rpt, rpt)],
                        accum.at[pl.ds(s * rpt, rpt)])
        plsc.subcore_barrier()
        descs = [
            pltpu.async_copy(buf.at[pl.ds(0, sz)],
                             accum.at[idxd.at[pl.ds(a, sz)]],
                             ssem, add=True)
            for a, sz in segs
        ]
        for d in descs:
            d.wait()
        plsc.subcore_barrier()
        pltpu.sync_copy(accum.at[pl.ds(s * rpt, rpt)],
                        out.at[c, pl.ds(s * rpt, rpt)])

    return pl.kernel(
        body,
        out_type=jax.ShapeDtypeStruct((NC, nrows, 16), jnp.float32),
        mesh=_mesh(),
        compiler_params=pltpu.CompilerParams(use_tc_tiling_on_sc=False),
        scratch_types=[
            pltpu.VMEM((ept,), jnp.int32),
            pltpu.VMEM((seg_max, 16), jnp.float32),
            pltpu.VMEM_SHARED((nrows, 16), jnp.float32),
            pltpu.SemaphoreType.DMA,
        ],
    )


def _make_agg(nrows, ept, f, nseg=NSEG):
    rpt = nrows // NS
    segs = _segments(ept, nseg)
    seg_max = max(sz for _, sz in segs)

    def body(srcs, dsts, g, zeros, out, idxs, idxd, buf0, buf1, accum,
             gsem, ssem):
        c = lax.axis_index("c")
        s = lax.axis_index("s")
        wid = s * NC + c
        bufs = [buf0, buf1]
        pltpu.sync_copy(srcs.at[wid], idxs)
        pltpu.sync_copy(dsts.at[wid], idxd)

        def gather(i):
            a, sz = segs[i]
            return pltpu.async_copy(
                g.at[idxs.at[pl.ds(a, sz)]],
                bufs[i % 2].at[pl.ds(0, sz)], gsem)

        def scatter(i):
            a, sz = segs[i]
            return pltpu.async_copy(
                bufs[i % 2].at[pl.ds(0, sz)],
                accum.at[idxd.at[pl.ds(a, sz)]],
                ssem, add=True)

        gd = gather(0)
        pltpu.sync_copy(zeros.at[pl.ds(s * rpt, rpt)],
                        accum.at[pl.ds(s * rpt, rpt)])
        plsc.subcore_barrier()

        sds = [None] * nseg
        for i in range(nseg):
            gd.wait()
            sds[i] = scatter(i)
            if i + 1 < nseg:
                if i >= 1:
                    sds[i - 1].wait()
                gd = gather(i + 1)
        for i in range(max(0, nseg - 2), nseg):
            sds[i].wait()

        plsc.subcore_barrier()
        pltpu.sync_copy(accum.at[pl.ds(s * rpt, rpt)],
                        out.at[c, pl.ds(s * rpt, rpt)])

    return pl.kernel(
        body,
        out_type=jax.ShapeDtypeStruct((NC, nrows, f), jnp.float32),
        mesh=_mesh(),
        compiler_params=pltpu.CompilerParams(use_tc_tiling_on_sc=False),
        scratch_types=[
            pltpu.VMEM((ept,), jnp.int32),
            pltpu.VMEM((ept,), jnp.int32),
            pltpu.VMEM((seg_max, f), jnp.float32),
            pltpu.VMEM((seg_max, f), jnp.float32),
            pltpu.VMEM_SHARED((nrows, f), jnp.float32),
            pltpu.SemaphoreType.DMA,
            pltpu.SemaphoreType.DMA,
        ],
    )


def _tc_a(n, br, deg_ref, x_ref, w1_ref, g1_ref, dinv_ref):
    i = pl.program_id(0)
    dp = deg_ref[...]
    deg = dp[0] + dp[1] + 1.0
    rows = i * br + lax.broadcasted_iota(jnp.int32, (br, 16), 0)
    dinv = jnp.where(rows < n, lax.rsqrt(deg), 0.0)
    xw = jnp.dot(x_ref[...], w1_ref[...], preferred_element_type=jnp.float32)
    g1_ref[...] = xw * dinv
    dinv_ref[...] = dinv


def _tc_b(s1_ref, g1_ref, dinv_ref, b1_ref, w2_ref, g2_ref):
    sp = s1_ref[...]
    dinv = dinv_ref[...]
    h1 = jnp.maximum((sp[0] + sp[1] + g1_ref[...]) * dinv + b1_ref[...], 0.0)
    g2_ref[...] = jnp.dot(h1, w2_ref[...],
                          preferred_element_type=jnp.float32) * dinv


def _tc_c(s2_ref, g2_ref, dinv_ref, b2_ref, w3_ref, h2_ref, g3_ref):
    sp = s2_ref[...]
    dinv = dinv_ref[...]
    hraw = (sp[0] + sp[1] + g2_ref[...]) * dinv + b2_ref[...]
    nrm = jnp.sqrt(jnp.sum(hraw * hraw, axis=1, keepdims=True))
    h2 = hraw / jnp.maximum(nrm, 1e-12)
    h2_ref[...] = h2
    g3_ref[...] = jnp.dot(h2, w3_ref[...],
                          preferred_element_type=jnp.float32) * dinv


def _tc_d(n, br, nb, s3_ref, g3_ref, dinv_ref, b3_ref, wp_ref, bp_ref,
          wl_ref, bl_ref, ls_ref, hp_ref, acc_ref):
    i = pl.program_id(0)
    sp = s3_ref[...]
    dinv = dinv_ref[...]
    h3 = (sp[0] + sp[1] + g3_ref[...]) * dinv + b3_ref[...]
    hp = jnp.dot(h3, wp_ref[...], preferred_element_type=jnp.float32) \
        + bp_ref[...]
    hp_ref[...] = hp
    rows = i * br + lax.broadcasted_iota(jnp.int32, (br, 16), 0)
    part = jnp.sum(jnp.where(rows < n, hp, 0.0), axis=0, keepdims=True)

    @pl.when(i == 0)
    def _():
        acc_ref[...] = part

    @pl.when(i > 0)
    def _():
        acc_ref[...] += part

    @pl.when(i == nb - 1)
    def _():
        pooled = acc_ref[...] * (1.0 / n)
        logits = jnp.dot(pooled, wl_ref[...],
                         preferred_element_type=jnp.float32) + bl_ref[...]
        m = jnp.max(logits, axis=1, keepdims=True)
        e2 = logits - m
        ls_ref[...] = e2 - jnp.log(jnp.sum(jnp.exp(e2), axis=1, keepdims=True))


def kernel(x, edge_index, W1, b1, W2, b2, W3, b3, Wp, bp, Wl, bl):
    f32 = jnp.float32
    n = x.shape[0]
    e = edge_index.shape[1]
    nrows = ((n + 1 + 127) // 128) * 128
    br = nrows // 4
    nb = nrows // br
    align = NW * 8
    epad = ((e + align - 1) // align) * align
    ept = epad // NW
    seg_max = max(sz for _, sz in _segments(ept, NSEG))

    ei = edge_index.astype(jnp.int32)
    if epad != e:
        ei = jnp.concatenate(
            [ei, jnp.full((2, epad - e), n, jnp.int32)], axis=1)
    src_p = ei[0].reshape(NW, ept)
    dst_p = ei[1].reshape(NW, ept)

    x_p = jnp.zeros((nrows, D), f32).at[:n].set(x)
    w2p = jnp.zeros((16, 16), f32).at[:, :3].set(W2)
    w3p = jnp.zeros((16, 16), f32).at[:3, :3].set(W3)
    wpp = jnp.zeros((16, 16), f32).at[:3, :3].set(Wp)
    wlp = jnp.zeros((16, Wl.shape[1]), f32).at[:3].set(Wl)
    b1r = b1.reshape(1, 16)
    b2p = jnp.zeros((1, 16), f32).at[0, :3].set(b2)
    b3p = jnp.zeros((1, 16), f32).at[0, :3].set(b3)
    bpp = jnp.zeros((1, 16), f32).at[0, :3].set(bp)
    blr = bl.reshape(1, -1)

    z16 = jnp.zeros((nrows, 16), f32)
    ones = jnp.ones((seg_max, 16), f32)

    row2 = lambda i: (0, i, 0)
    row = lambda i: (i, 0)
    fixed = lambda i: (0, 0)
    sp2 = pl.BlockSpec((2, br, 16), row2)
    spg = pl.BlockSpec((br, 16), row)

    degp = _make_deg(nrows, ept)(dst_p, z16, ones)

    g1, dinv = pl.pallas_call(
        functools.partial(_tc_a, n, br),
        grid=(nb,),
        in_specs=[sp2, pl.BlockSpec((br, D), row),
                  pl.BlockSpec((D, 16), fixed)],
        out_specs=[spg, spg],
        out_shape=[jax.ShapeDtypeStruct((nrows, 16), f32),
                   jax.ShapeDtypeStruct((nrows, 16), f32)],
    )(degp, x_p, W1)

    s1 = _make_agg(nrows, ept, 16)(src_p, dst_p, g1, z16)

    g2 = pl.pallas_call(
        _tc_b,
        grid=(nb,),
        in_specs=[sp2, spg, spg, pl.BlockSpec((1, 16), fixed),
                  pl.BlockSpec((16, 16), fixed)],
        out_specs=spg,
        out_shape=jax.ShapeDtypeStruct((nrows, 16), f32),
    )(s1, g1, dinv, b1r, w2p)

    s2 = _make_agg(nrows, ept, 16)(src_p, dst_p, g2, z16)

    h2, g3 = pl.pallas_call(
        _tc_c,
        grid=(nb,),
        in_specs=[sp2, spg, spg, pl.BlockSpec((1, 16), fixed),
                  pl.BlockSpec((16, 16), fixed)],
        out_specs=[spg, spg],
        out_shape=[jax.ShapeDtypeStruct((nrows, 16), f32),
                   jax.ShapeDtypeStruct((nrows, 16), f32)],
    )(s2, g2, dinv, b2p, w3p)

    s3 = _make_agg(nrows, ept, 16)(src_p, dst_p, g3, z16)

    ls, hp = pl.pallas_call(
        functools.partial(_tc_d, n, br, nb),
        grid=(nb,),
        in_specs=[sp2, spg, spg, pl.BlockSpec((1, 16), fixed),
                  pl.BlockSpec((16, 16), fixed),
                  pl.BlockSpec((1, 16), fixed),
                  pl.BlockSpec((16, Wl.shape[1]), fixed),
                  pl.BlockSpec((1, Wl.shape[1]), fixed)],
        out_specs=[pl.BlockSpec((1, Wl.shape[1]), fixed), spg],
        out_shape=[jax.ShapeDtypeStruct((1, Wl.shape[1]), f32),
                   jax.ShapeDtypeStruct((nrows, 16), f32)],
        scratch_shapes=[pltpu.VMEM((1, 16), f32)],
    )(s3, g3, dinv, b3p, wpp, bpp, wlp, blr)

    return (ls, (hp[:n, :3], h2[:n, :3]))


# speedup vs baseline: 1.1402x; 1.0067x over previous
import functools

import jax
import jax.numpy as jnp
from jax import lax
from jax.experimental import pallas as pl
from jax.experimental.pallas import tpu as pltpu
from jax.experimental.pallas import tpu_sc as plsc

NC = 2
NS = 16
NW = NC * NS
NSEG = 4
D = 128


def _mesh():
    return plsc.VectorSubcoreMesh(
        core_axis_name="c", subcore_axis_name="s",
        num_cores=NC, num_subcores=NS)


def _segments(total, nseg):
    bounds = [((total * i // nseg) // 8) * 8 for i in range(nseg + 1)]
    bounds[-1] = total
    return [(bounds[i], bounds[i + 1] - bounds[i]) for i in range(nseg)]


def _make_deg(nrows, ept, nseg=NSEG):
    rpt = nrows // NS
    segs = _segments(ept, nseg)
    seg_max = max(sz for _, sz in segs)

    def body(dsts, zeros, ones, out, idxd, buf, accum, ssem):
        c = lax.axis_index("c")
        s = lax.axis_index("s")
        wid = s * NC + c
        pltpu.sync_copy(dsts.at[wid], idxd)
        pltpu.sync_copy(ones, buf)
        pltpu.sync_copy(zeros.at[pl.ds(s * rpt, rpt)],
                        accum.at[pl.ds(s * rpt, rpt)])
        plsc.subcore_barrier()
        descs = [
            pltpu.async_copy(buf.at[pl.ds(0, sz)],
                             accum.at[idxd.at[pl.ds(a, sz)]],
                             ssem, add=True)
            for a, sz in segs
        ]
        for d in descs:
            d.wait()
        plsc.subcore_barrier()
        pltpu.sync_copy(accum.at[pl.ds(s * rpt, rpt)],
                        out.at[c, pl.ds(s * rpt, rpt)])

    return pl.kernel(
        body,
        out_type=jax.ShapeDtypeStruct((NC, nrows, 16), jnp.float32),
        mesh=_mesh(),
        compiler_params=pltpu.CompilerParams(use_tc_tiling_on_sc=False),
        scratch_types=[
            pltpu.VMEM((ept,), jnp.int32),
            pltpu.VMEM((seg_max, 16), jnp.float32),
            pltpu.VMEM_SHARED((nrows, 16), jnp.float32),
            pltpu.SemaphoreType.DMA,
        ],
    )


def _make_agg(nrows, ept, f, nseg=NSEG):
    rpt = nrows // NS
    segs = _segments(ept, nseg)
    seg_max = max(sz for _, sz in segs)

    def body(srcs, dsts, g, zeros, out, idxs, idxd, buf0, buf1, accum,
             gsem, ssem):
        c = lax.axis_index("c")
        s = lax.axis_index("s")
        wid = s * NC + c
        bufs = [buf0, buf1]
        pltpu.sync_copy(srcs.at[wid], idxs)
        pltpu.sync_copy(dsts.at[wid], idxd)

        def gather(i):
            a, sz = segs[i]
            return pltpu.async_copy(
                g.at[idxs.at[pl.ds(a, sz)]],
                bufs[i % 2].at[pl.ds(0, sz)], gsem)

        def scatter(i):
            a, sz = segs[i]
            return pltpu.async_copy(
                bufs[i % 2].at[pl.ds(0, sz)],
                accum.at[idxd.at[pl.ds(a, sz)]],
                ssem, add=True)

        gd = gather(0)
        pltpu.sync_copy(zeros.at[pl.ds(s * rpt, rpt)],
                        accum.at[pl.ds(s * rpt, rpt)])
        plsc.subcore_barrier()

        sds = [None] * nseg
        for i in range(nseg):
            gd.wait()
            sds[i] = scatter(i)
            if i + 1 < nseg:
                if i >= 1:
                    sds[i - 1].wait()
                gd = gather(i + 1)
        for i in range(max(0, nseg - 2), nseg):
            sds[i].wait()

        plsc.subcore_barrier()
        pltpu.sync_copy(accum.at[pl.ds(s * rpt, rpt)],
                        out.at[c, pl.ds(s * rpt, rpt)])

    return pl.kernel(
        body,
        out_type=jax.ShapeDtypeStruct((NC, nrows, f), jnp.float32),
        mesh=_mesh(),
        compiler_params=pltpu.CompilerParams(use_tc_tiling_on_sc=False),
        scratch_types=[
            pltpu.VMEM((ept,), jnp.int32),
            pltpu.VMEM((ept,), jnp.int32),
            pltpu.VMEM((seg_max, f), jnp.float32),
            pltpu.VMEM((seg_max, f), jnp.float32),
            pltpu.VMEM_SHARED((nrows, f), jnp.float32),
            pltpu.SemaphoreType.DMA,
            pltpu.SemaphoreType.DMA,
        ],
    )


def _tc_a(n, br, deg_ref, x_ref, w1_ref, g1_ref, dinv_ref):
    i = pl.program_id(0)
    dp = deg_ref[...]
    deg = dp[0] + dp[1] + 1.0
    rows = i * br + lax.broadcasted_iota(jnp.int32, (br, 16), 0)
    dinv = jnp.where(rows < n, lax.rsqrt(deg), 0.0)
    xw = jnp.dot(x_ref[...], w1_ref[...], preferred_element_type=jnp.float32)
    g1_ref[...] = xw * dinv
    dinv_ref[...] = dinv


def _tc_b(s1_ref, g1_ref, dinv_ref, b1_ref, w2_ref, g2_ref):
    sp = s1_ref[...]
    dinv = dinv_ref[...]
    h1 = jnp.maximum((sp[0] + sp[1] + g1_ref[...]) * dinv + b1_ref[...], 0.0)
    g2_ref[...] = jnp.dot(h1, w2_ref[...],
                          preferred_element_type=jnp.float32) * dinv


def _tc_c(s2_ref, g2_ref, dinv_ref, b2_ref, w3_ref, h2_ref, g3_ref):
    sp = s2_ref[...]
    dinv = dinv_ref[...]
    hraw = (sp[0] + sp[1] + g2_ref[...]) * dinv + b2_ref[...]
    nrm = jnp.sqrt(jnp.sum(hraw * hraw, axis=1, keepdims=True))
    h2 = hraw / jnp.maximum(nrm, 1e-12)
    h2_ref[...] = h2
    g3_ref[...] = jnp.dot(h2, w3_ref[...],
                          preferred_element_type=jnp.float32) * dinv


def _tc_d(n, br, nb, s3_ref, g3_ref, dinv_ref, b3_ref, wp_ref, bp_ref,
          wl_ref, bl_ref, ls_ref, hp_ref, acc_ref):
    i = pl.program_id(0)
    sp = s3_ref[...]
    dinv = dinv_ref[...]
    h3 = (sp[0] + sp[1] + g3_ref[...]) * dinv + b3_ref[...]
    hp = jnp.dot(h3, wp_ref[...], preferred_element_type=jnp.float32) \
        + bp_ref[...]
    hp_ref[...] = hp
    rows = i * br + lax.broadcasted_iota(jnp.int32, (br, 16), 0)
    part = jnp.sum(jnp.where(rows < n, hp, 0.0), axis=0, keepdims=True)

    @pl.when(i == 0)
    def _():
        acc_ref[...] = part

    @pl.when(i > 0)
    def _():
        acc_ref[...] += part

    @pl.when(i == nb - 1)
    def _():
        pooled = acc_ref[...] * (1.0 / n)
        logits = jnp.dot(pooled, wl_ref[...],
                         preferred_element_type=jnp.float32) + bl_ref[...]
        m = jnp.max(logits, axis=1, keepdims=True)
        e2 = logits - m
        ls_ref[...] = e2 - jnp.log(jnp.sum(jnp.exp(e2), axis=1, keepdims=True))


def kernel(x, edge_index, W1, b1, W2, b2, W3, b3, Wp, bp, Wl, bl):
    f32 = jnp.float32
    n = x.shape[0]
    e = edge_index.shape[1]
    nrows = ((n + 1 + 127) // 128) * 128
    br = nrows // 2
    nb = nrows // br
    align = NW * 8
    epad = ((e + align - 1) // align) * align
    ept = epad // NW
    seg_max = max(sz for _, sz in _segments(ept, NSEG))

    ei = edge_index.astype(jnp.int32)
    if epad != e:
        ei = jnp.concatenate(
            [ei, jnp.full((2, epad - e), n, jnp.int32)], axis=1)
    src_p = ei[0].reshape(NW, ept)
    dst_p = ei[1].reshape(NW, ept)

    x_p = jnp.zeros((nrows, D), f32).at[:n].set(x)
    w2p = jnp.zeros((16, 16), f32).at[:, :3].set(W2)
    w3p = jnp.zeros((16, 16), f32).at[:3, :3].set(W3)
    wpp = jnp.zeros((16, 16), f32).at[:3, :3].set(Wp)
    wlp = jnp.zeros((16, Wl.shape[1]), f32).at[:3].set(Wl)
    b1r = b1.reshape(1, 16)
    b2p = jnp.zeros((1, 16), f32).at[0, :3].set(b2)
    b3p = jnp.zeros((1, 16), f32).at[0, :3].set(b3)
    bpp = jnp.zeros((1, 16), f32).at[0, :3].set(bp)
    blr = bl.reshape(1, -1)

    z16 = jnp.zeros((nrows, 16), f32)
    ones = jnp.ones((seg_max, 16), f32)

    row2 = lambda i: (0, i, 0)
    row = lambda i: (i, 0)
    fixed = lambda i: (0, 0)
    sp2 = pl.BlockSpec((2, br, 16), row2)
    spg = pl.BlockSpec((br, 16), row)

    degp = _make_deg(nrows, ept)(dst_p, z16, ones)

    g1, dinv = pl.pallas_call(
        functools.partial(_tc_a, n, br),
        grid=(nb,),
        in_specs=[sp2, pl.BlockSpec((br, D), row),
                  pl.BlockSpec((D, 16), fixed)],
        out_specs=[spg, spg],
        out_shape=[jax.ShapeDtypeStruct((nrows, 16), f32),
                   jax.ShapeDtypeStruct((nrows, 16), f32)],
    )(degp, x_p, W1)

    s1 = _make_agg(nrows, ept, 16)(src_p, dst_p, g1, z16)

    g2 = pl.pallas_call(
        _tc_b,
        grid=(nb,),
        in_specs=[sp2, spg, spg, pl.BlockSpec((1, 16), fixed),
                  pl.BlockSpec((16, 16), fixed)],
        out_specs=spg,
        out_shape=jax.ShapeDtypeStruct((nrows, 16), f32),
    )(s1, g1, dinv, b1r, w2p)

    s2 = _make_agg(nrows, ept, 16)(src_p, dst_p, g2, z16)

    h2, g3 = pl.pallas_call(
        _tc_c,
        grid=(nb,),
        in_specs=[sp2, spg, spg, pl.BlockSpec((1, 16), fixed),
                  pl.BlockSpec((16, 16), fixed)],
        out_specs=[spg, spg],
        out_shape=[jax.ShapeDtypeStruct((nrows, 16), f32),
                   jax.ShapeDtypeStruct((nrows, 16), f32)],
    )(s2, g2, dinv, b2p, w3p)

    s3 = _make_agg(nrows, ept, 16)(src_p, dst_p, g3, z16)

    ls, hp = pl.pallas_call(
        functools.partial(_tc_d, n, br, nb),
        grid=(nb,),
        in_specs=[sp2, spg, spg, pl.BlockSpec((1, 16), fixed),
                  pl.BlockSpec((16, 16), fixed),
                  pl.BlockSpec((1, 16), fixed),
                  pl.BlockSpec((16, Wl.shape[1]), fixed),
                  pl.BlockSpec((1, Wl.shape[1]), fixed)],
        out_specs=[pl.BlockSpec((1, Wl.shape[1]), fixed), spg],
        out_shape=[jax.ShapeDtypeStruct((1, Wl.shape[1]), f32),
                   jax.ShapeDtypeStruct((nrows, 16), f32)],
        scratch_shapes=[pltpu.VMEM((1, 16), f32)],
    )(s3, g3, dinv, b3p, wpp, bpp, wlp, blr)

    return (ls, (hp[:n, :3], h2[:n, :3]))
